# baseline (speedup 1.0000x reference)
import functools

import jax
import jax.numpy as jnp
from jax import lax
from jax.experimental import pallas as pl
from jax.experimental.pallas import tpu as pltpu
from jax.experimental.pallas import tpu_sc as plsc

_NC = 2
_NS = 16
_NW = _NC * _NS

_EPS = 1.1920929e-07


def _make_sc_gather_sum(V, D, N, K, CH=8, NBUF=4):
    rows_per_w = N // _NW
    NCH = rows_per_w // CH
    mesh = plsc.VectorSubcoreMesh(core_axis_name="c", subcore_axis_name="s")

    @functools.partial(
        pl.kernel,
        mesh=mesh,
        out_type=jax.ShapeDtypeStruct((N, D), jnp.float32),
        scratch_types=[
            pltpu.VMEM((NCH, CH * K), jnp.int32),
            pltpu.VMEM((NBUF, CH * K, D), jnp.float32),
            pltpu.VMEM((NBUF, CH, D), jnp.float32),
        ] + [pltpu.SemaphoreType.DMA] * (2 * NBUF),
    )
    def sc_gather_sum(table_hbm, idx_hbm, out_hbm, idx_v, gbuf, sbuf, *sems):
        gsem = sems[:NBUF]
        osem = sems[NBUF:]
        wid = lax.axis_index("s") * _NC + lax.axis_index("c")
        base = wid * rows_per_w
        pltpu.sync_copy(idx_hbm.at[wid], idx_v)
        for b in range(NBUF):
            pltpu.async_copy(table_hbm.at[idx_v.at[b]], gbuf.at[b], gsem[b])

        def ring(i, carry):
            c0 = i * NBUF
            for b in range(NBUF):
                c = c0 + b
                pltpu.make_async_copy(
                    table_hbm.at[idx_v.at[c]], gbuf.at[b], gsem[b]).wait()
                @pl.when(c >= NBUF)
                def _():
                    pltpu.make_async_copy(
                        sbuf.at[b], out_hbm.at[pl.ds(base, CH)],
                        osem[b]).wait()

                @plsc.parallel_loop(0, CH)
                def _rloop(r):
                    for j in range(D // 16):
                        o = j * 16
                        s = (gbuf[b, K * r, pl.ds(o, 16)]
                             + gbuf[b, K * r + 1, pl.ds(o, 16)]
                             + gbuf[b, K * r + 2, pl.ds(o, 16)]
                             + gbuf[b, K * r + 3, pl.ds(o, 16)])
                        sbuf[b, r, pl.ds(o, 16)] = s
                pltpu.async_copy(
                    sbuf.at[b], out_hbm.at[pl.ds(base + c * CH, CH)],
                    osem[b])

                @pl.when(c + NBUF < NCH)
                def _():
                    pltpu.async_copy(
                        table_hbm.at[idx_v.at[c + NBUF]], gbuf.at[b], gsem[b])
            return carry

        lax.fori_loop(0, NCH // NBUF, ring, 0)
        for b in range(NBUF):
            pltpu.make_async_copy(
                sbuf.at[b], out_hbm.at[pl.ds(base, CH)], osem[b]).wait()

    return sc_gather_sum


def _tc_body(h_ref, e_ref, wk_ref, wkg_ref, wv_ref, wvg_ref,
             cw_ref, cb_ref, o_ref, wkq_s, wks_s, wvq_s, wvs_s, hprev):
    b = pl.program_id(0)
    i = pl.program_id(1)

    @pl.when(jnp.logical_and(b == 0, i == 0))
    def _():
        wk = wk_ref[...]
        sk = jnp.clip(jnp.mean(jnp.abs(wk)), 1e-5, None)
        wkq_s[...] = jnp.clip(jnp.round(wk * (1.0 / sk)), -1,
                              1).astype(jnp.bfloat16)
        wks_s[...] = sk.reshape(1, 1)
        wv = wv_ref[...]
        sv = jnp.clip(jnp.mean(jnp.abs(wv)), 1e-5, None)
        wvq_s[...] = jnp.clip(jnp.round(wv * (1.0 / sv)), -1,
                              1).astype(jnp.bfloat16)
        wvs_s[...] = sv.reshape(1, 1)

    e = e_ref[0]
    h = h_ref[0]
    halo = jnp.where(i > 0, hprev[...], 0.0)
    z1 = jnp.concatenate([halo[1:2], e[:-1]], axis=0)
    z2 = jnp.concatenate([halo[0:2], e[:-2]], axis=0)
    hprev[...] = e[-2:]
    e_conv = (z2 * cw_ref[0:1] + z1 * cw_ref[1:2] + e * cw_ref[2:3]
              + cb_ref[...])

    rms_e = e_conv * lax.rsqrt(
        jnp.mean(e_conv * e_conv, axis=-1, keepdims=True) + _EPS)
    q_norm = h * lax.rsqrt(jnp.mean(h * h, axis=-1, keepdims=True) + _EPS)

    def bitlinear(g_row, wq, sinv):
        xn = rms_e * g_row
        amax = jnp.clip(jnp.max(jnp.abs(xn), axis=-1, keepdims=True),
                        1e-5, None)
        xqi = jnp.round(xn * (127.0 / amax))
        acc = lax.dot_general(xqi.astype(jnp.bfloat16), wq,
                              (((1,), (1,)), ((), ())),
                              preferred_element_type=jnp.float32)
        return acc * (amax * ((1.0 / 127.0) * sinv))

    k = bitlinear(wkg_ref[...], wkq_s[...], wks_s[0, 0])
    k_norm = k * lax.rsqrt(jnp.mean(k * k, axis=-1, keepdims=True) + _EPS)
    sim = jnp.sum(q_norm * k_norm, axis=-1, keepdims=True)
    alpha = jax.nn.sigmoid(sim)
    v = bitlinear(wvg_ref[...], wvq_s[...], wvs_s[0, 0])
    o_ref[0] = h + alpha * v


def kernel(h_t, memory_table, Wk_w, Wk_g, Wv_w, Wv_g, conv_w, conv_b,
           hash_ngrams):
    B, L, D = h_t.shape
    K = hash_ngrams.shape[-1]
    V = memory_table.shape[0]
    N = B * L
    rows_per_w = N // _NW
    CH = 8
    NCH = rows_per_w // CH

    idx = hash_ngrams.astype(jnp.int32).reshape(_NW, NCH, CH * K)
    e_t = _make_sc_gather_sum(V, D, N, K)(memory_table, idx).reshape(B, L, D)

    TL = 512
    nblk = L // TL

    spec_bld = pl.BlockSpec((1, TL, D), lambda b, i: (b, i, 0))
    spec_w = pl.BlockSpec((D, D), lambda b, i: (0, 0))
    spec_row = pl.BlockSpec((1, D), lambda b, i: (0, 0))
    spec_cw = pl.BlockSpec((3, D), lambda b, i: (0, 0))

    out = pl.pallas_call(
        _tc_body,
        grid=(B, nblk),
        in_specs=[spec_bld, spec_bld, spec_w, spec_row,
                  spec_w, spec_row, spec_cw, spec_row],
        out_specs=spec_bld,
        out_shape=jax.ShapeDtypeStruct((B, L, D), jnp.float32),
        scratch_shapes=[
            pltpu.VMEM((D, D), jnp.bfloat16),
            pltpu.VMEM((1, 1), jnp.float32),
            pltpu.VMEM((D, D), jnp.bfloat16),
            pltpu.VMEM((1, 1), jnp.float32),
            pltpu.VMEM((2, D), jnp.float32),
        ],
    )(h_t, e_t, Wk_w, Wk_g.reshape(1, D),
      Wv_w, Wv_g.reshape(1, D), conv_w.T, conv_b.reshape(1, D))
    return out


# speedup vs baseline: 1.2732x; 1.2732x over previous
import functools

import jax
import jax.numpy as jnp
from jax import lax
from jax.experimental import pallas as pl
from jax.experimental.pallas import tpu as pltpu
from jax.experimental.pallas import tpu_sc as plsc

_NC = 2
_NS = 16
_NW = _NC * _NS

_EPS = 1.1920929e-07


def _make_sc_gather_sum(V, D, N, K, CH=16, NBUF=2):
    rows_per_w = N // _NW
    NCH = rows_per_w // CH
    mesh = plsc.VectorSubcoreMesh(core_axis_name="c", subcore_axis_name="s")

    @functools.partial(
        pl.kernel,
        mesh=mesh,
        out_type=jax.ShapeDtypeStruct((N, D), jnp.float32),
        scratch_types=[
            pltpu.VMEM((NCH, CH * K), jnp.int32),
            pltpu.VMEM((NBUF, CH * K, D), jnp.float32),
            pltpu.VMEM((NBUF, CH, D), jnp.float32),
        ] + [pltpu.SemaphoreType.DMA] * (2 * NBUF),
    )
    def sc_gather_sum(table_hbm, idx_hbm, out_hbm, idx_v, gbuf, sbuf, *sems):
        gsem = sems[:NBUF]
        osem = sems[NBUF:]
        wid = lax.axis_index("s") * _NC + lax.axis_index("c")
        base = wid * rows_per_w
        pltpu.sync_copy(idx_hbm.at[wid], idx_v)
        for b in range(NBUF):
            pltpu.async_copy(table_hbm.at[idx_v.at[b]], gbuf.at[b], gsem[b])

        def ring(i, carry):
            c0 = i * NBUF
            for b in range(NBUF):
                c = c0 + b
                pltpu.make_async_copy(
                    table_hbm.at[idx_v.at[c]], gbuf.at[b], gsem[b]).wait()
                @pl.when(c >= NBUF)
                def _():
                    pltpu.make_async_copy(
                        sbuf.at[b], out_hbm.at[pl.ds(base, CH)],
                        osem[b]).wait()

                @plsc.parallel_loop(0, CH)
                def _rloop(r):
                    for j in range(D // 16):
                        o = j * 16
                        s = (gbuf[b, K * r, pl.ds(o, 16)]
                             + gbuf[b, K * r + 1, pl.ds(o, 16)]
                             + gbuf[b, K * r + 2, pl.ds(o, 16)]
                             + gbuf[b, K * r + 3, pl.ds(o, 16)])
                        sbuf[b, r, pl.ds(o, 16)] = s
                pltpu.async_copy(
                    sbuf.at[b], out_hbm.at[pl.ds(base + c * CH, CH)],
                    osem[b])

                @pl.when(c + NBUF < NCH)
                def _():
                    pltpu.async_copy(
                        table_hbm.at[idx_v.at[c + NBUF]], gbuf.at[b], gsem[b])
            return carry

        lax.fori_loop(0, NCH // NBUF, ring, 0)
        for b in range(NBUF):
            pltpu.make_async_copy(
                sbuf.at[b], out_hbm.at[pl.ds(base, CH)], osem[b]).wait()

    return sc_gather_sum


def _tc_body(h_ref, e_ref, wk_ref, wkg_ref, wv_ref, wvg_ref,
             cw_ref, cb_ref, o_ref, wkq_s, wks_s, wvq_s, wvs_s, hprev):
    b = pl.program_id(0)
    i = pl.program_id(1)

    @pl.when(jnp.logical_and(b == 0, i == 0))
    def _():
        wk = wk_ref[...]
        sk = jnp.clip(jnp.mean(jnp.abs(wk)), 1e-5, None)
        wkq_s[...] = jnp.clip(jnp.round(wk * (1.0 / sk)), -1,
                              1).astype(jnp.bfloat16)
        wks_s[...] = sk.reshape(1, 1)
        wv = wv_ref[...]
        sv = jnp.clip(jnp.mean(jnp.abs(wv)), 1e-5, None)
        wvq_s[...] = jnp.clip(jnp.round(wv * (1.0 / sv)), -1,
                              1).astype(jnp.bfloat16)
        wvs_s[...] = sv.reshape(1, 1)

    e = e_ref[0]
    h = h_ref[0]
    halo = jnp.where(i > 0, hprev[...], 0.0)
    z1 = jnp.concatenate([halo[1:2], e[:-1]], axis=0)
    z2 = jnp.concatenate([halo[0:2], e[:-2]], axis=0)
    hprev[...] = e[-2:]
    e_conv = (z2 * cw_ref[0:1] + z1 * cw_ref[1:2] + e * cw_ref[2:3]
              + cb_ref[...])

    rms_e = e_conv * lax.rsqrt(
        jnp.mean(e_conv * e_conv, axis=-1, keepdims=True) + _EPS)
    q_norm = h * lax.rsqrt(jnp.mean(h * h, axis=-1, keepdims=True) + _EPS)

    def bitlinear(g_row, wq, sinv):
        xn = rms_e * g_row
        amax = jnp.clip(jnp.max(jnp.abs(xn), axis=-1, keepdims=True),
                        1e-5, None)
        xqi = jnp.round(xn * (127.0 / amax))
        acc = lax.dot_general(xqi.astype(jnp.bfloat16), wq,
                              (((1,), (1,)), ((), ())),
                              preferred_element_type=jnp.float32)
        return acc * (amax * ((1.0 / 127.0) * sinv))

    k = bitlinear(wkg_ref[...], wkq_s[...], wks_s[0, 0])
    k_norm = k * lax.rsqrt(jnp.mean(k * k, axis=-1, keepdims=True) + _EPS)
    sim = jnp.sum(q_norm * k_norm, axis=-1, keepdims=True)
    alpha = jax.nn.sigmoid(sim)
    v = bitlinear(wvg_ref[...], wvq_s[...], wvs_s[0, 0])
    o_ref[0] = h + alpha * v


def kernel(h_t, memory_table, Wk_w, Wk_g, Wv_w, Wv_g, conv_w, conv_b,
           hash_ngrams):
    B, L, D = h_t.shape
    K = hash_ngrams.shape[-1]
    V = memory_table.shape[0]
    N = B * L
    rows_per_w = N // _NW
    CH = 16
    NCH = rows_per_w // CH

    idx = hash_ngrams.astype(jnp.int32).reshape(_NW, NCH, CH * K)
    e_t = _make_sc_gather_sum(V, D, N, K)(memory_table, idx).reshape(B, L, D)

    TL = 512
    nblk = L // TL

    spec_bld = pl.BlockSpec((1, TL, D), lambda b, i: (b, i, 0))
    spec_w = pl.BlockSpec((D, D), lambda b, i: (0, 0))
    spec_row = pl.BlockSpec((1, D), lambda b, i: (0, 0))
    spec_cw = pl.BlockSpec((3, D), lambda b, i: (0, 0))

    out = pl.pallas_call(
        _tc_body,
        grid=(B, nblk),
        in_specs=[spec_bld, spec_bld, spec_w, spec_row,
                  spec_w, spec_row, spec_cw, spec_row],
        out_specs=spec_bld,
        out_shape=jax.ShapeDtypeStruct((B, L, D), jnp.float32),
        scratch_shapes=[
            pltpu.VMEM((D, D), jnp.bfloat16),
            pltpu.VMEM((1, 1), jnp.float32),
            pltpu.VMEM((D, D), jnp.bfloat16),
            pltpu.VMEM((1, 1), jnp.float32),
            pltpu.VMEM((2, D), jnp.float32),
        ],
    )(h_t, e_t, Wk_w, Wk_g.reshape(1, D),
      Wv_w, Wv_g.reshape(1, D), conv_w.T, conv_b.reshape(1, D))
    return out
